# baseline (speedup 1.0000x reference)
import functools

import jax
import jax.numpy as jnp
from jax import lax
from jax.experimental import pallas as pl
from jax.experimental.pallas import tpu as pltpu
from jax.experimental.pallas import tpu_sc as plsc

VOCAB = 100000
EMB = 128
BATCH = 4096
HIST = 200
B = BATCH * HIST

NC = 2
NS = 16
NW = NC * NS
B_PER_W = B // NW
CHUNK = 64
NCHUNK = B_PER_W // CHUNK
M = 10
K = 5

_mesh = plsc.VectorSubcoreMesh(core_axis_name="c", subcore_axis_name="s")


@functools.partial(
    pl.kernel,
    mesh=_mesh,
    out_type=jax.ShapeDtypeStruct((B, EMB), jnp.float32),
    scratch_types=[
        pltpu.VMEM((B_PER_W,), jnp.int32),
        pltpu.VMEM((M, CHUNK, EMB), jnp.float32),
        pltpu.SemaphoreType.DMA((M,)),
        pltpu.SemaphoreType.DMA((M,)),
    ],
)
def _gather(idx_hbm, table_hbm, out_hbm, idx_v, rows_v, gsem, ssem):
    wid = lax.axis_index("s") * NC + lax.axis_index("c")
    base = wid * B_PER_W

    pltpu.sync_copy(idx_hbm.at[pl.ds(base, B_PER_W)], idx_v)

    @pl.when(wid % 2 == 0)
    def _active():
        for b in range(M):
            pltpu.async_copy(
                table_hbm.at[idx_v.at[pl.ds(b * CHUNK, CHUNK)]],
                rows_v.at[b], gsem.at[b])

        def outer(i, carry):
            for b in range(M):
                c = i * M + b
                pltpu.make_async_copy(
                    table_hbm.at[idx_v.at[pl.ds(b * CHUNK, CHUNK)]],
                    rows_v.at[b], gsem.at[b]).wait()

                @pl.when(c + M < 2 * NCHUNK)
                def _refill():
                    pltpu.async_copy(
                        table_hbm.at[idx_v.at[pl.ds((c % 2) * CHUNK, CHUNK)]],
                        rows_v.at[b], gsem.at[b])
            return carry

        lax.fori_loop(0, 2 * NCHUNK // M, outer, 0)


def kernel(inputs, embedding):
    idx = inputs.reshape(-1).astype(jnp.int32)
    out = _gather(idx, embedding)
    return out.reshape(BATCH, HIST, EMB)


# speedup vs baseline: 1.3457x; 1.3457x over previous
import functools

import jax
import jax.numpy as jnp
from jax import lax
from jax.experimental import pallas as pl
from jax.experimental.pallas import tpu as pltpu
from jax.experimental.pallas import tpu_sc as plsc

VOCAB = 100000
EMB = 128
BATCH = 4096
HIST = 200
B = BATCH * HIST

NC = 2
NS = 16
NW = NC * NS
B_PER_W = B // NW
CHUNK = 64
NCHUNK = B_PER_W // CHUNK
M = 10
K = 5

_mesh = plsc.VectorSubcoreMesh(core_axis_name="c", subcore_axis_name="s")


@functools.partial(
    pl.kernel,
    mesh=_mesh,
    out_type=jax.ShapeDtypeStruct((B, EMB), jnp.float32),
    scratch_types=[
        pltpu.VMEM((B_PER_W,), jnp.int32),
        pltpu.VMEM((M, CHUNK, EMB), jnp.float32),
        pltpu.SemaphoreType.DMA((M,)),
        pltpu.SemaphoreType.DMA((M,)),
    ],
)
def _gather(idx_hbm, table_hbm, out_hbm, idx_v, rows_v, gsem, ssem):
    wid = lax.axis_index("s") * NC + lax.axis_index("c")
    base = wid * B_PER_W

    pltpu.sync_copy(idx_hbm.at[pl.ds(base, B_PER_W)], idx_v)
    for b in range(K):
        pltpu.async_copy(
            table_hbm.at[idx_v.at[pl.ds(b * CHUNK, CHUNK)]],
            rows_v.at[b], gsem.at[b])

    def outer(i, carry):
        for b in range(M):
            c = i * M + b
            off = base + c * CHUNK
            bn = (b + K) % M
            pltpu.make_async_copy(
                table_hbm.at[idx_v.at[pl.ds(b * CHUNK, CHUNK)]],
                rows_v.at[b], gsem.at[b]).wait()
            pltpu.async_copy(
                rows_v.at[b], out_hbm.at[pl.ds(off, CHUNK)], ssem.at[b])

            @pl.when(c + K < NCHUNK)
            def _refill():
                @pl.when(c >= M - K)
                def _wait_store():
                    pltpu.make_async_copy(
                        rows_v.at[bn],
                        out_hbm.at[pl.ds(off + (K - M) * CHUNK, CHUNK)],
                        ssem.at[bn]).wait()
                pltpu.async_copy(
                    table_hbm.at[idx_v.at[pl.ds((c + K) * CHUNK, CHUNK)]],
                    rows_v.at[bn], gsem.at[bn])
        return carry

    lax.fori_loop(0, NCHUNK // M, outer, 0)

    for x in range(NCHUNK - M, NCHUNK):
        pltpu.make_async_copy(
            rows_v.at[x % M],
            out_hbm.at[pl.ds(base + x * CHUNK, CHUNK)],
            ssem.at[x % M]).wait()


def kernel(inputs, embedding):
    idx = inputs.reshape(-1).astype(jnp.int32)
    out = _gather(idx, embedding)
    return out.reshape(BATCH, HIST, EMB)


# speedup vs baseline: 1.3643x; 1.0139x over previous
import functools

import jax
import jax.numpy as jnp
from jax import lax
from jax.experimental import pallas as pl
from jax.experimental.pallas import tpu as pltpu
from jax.experimental.pallas import tpu_sc as plsc

VOCAB = 100000
EMB = 128
BATCH = 4096
HIST = 200
B = BATCH * HIST

NC = 2
NS = 16
NW = NC * NS
B_PER_W = B // NW
CHUNK = 80
NCHUNK = B_PER_W // CHUNK
M = 4
S = 4
D = 2

_mesh = plsc.VectorSubcoreMesh(core_axis_name="c", subcore_axis_name="s")


@functools.partial(
    pl.kernel,
    mesh=_mesh,
    out_type=jax.ShapeDtypeStruct((B, EMB), jnp.float32),
    scratch_types=[
        pltpu.VMEM((B_PER_W,), jnp.int32),
        pltpu.VMEM((M, CHUNK, EMB), jnp.float32),
        pltpu.VMEM_SHARED((NS, S, CHUNK, EMB), jnp.float32),
        pltpu.SemaphoreType.DMA((M,)),
        pltpu.SemaphoreType.DMA((S,)),
        pltpu.SemaphoreType.DMA((S,)),
    ],
)
def _gather(idx_hbm, table_hbm, out_hbm, idx_v, rows_v, shared, gsem, psem,
            osem):
    cid = lax.axis_index("c")
    sid = lax.axis_index("s")
    wid = sid * NC + cid
    base = wid * B_PER_W

    pltpu.sync_copy(idx_hbm.at[pl.ds(base, B_PER_W)], idx_v)
    for b in range(M):
        pltpu.async_copy(
            table_hbm.at[idx_v.at[pl.ds(b * CHUNK, CHUNK)]],
            rows_v.at[b], gsem.at[b])

    def outer(i, carry):
        for u in range(M):
            c = i * M + u
            off = base + c * CHUNK
            pltpu.make_async_copy(
                table_hbm.at[idx_v.at[pl.ds(u * CHUNK, CHUNK)]],
                rows_v.at[u], gsem.at[u]).wait()

            @pl.when(c >= S)
            def _slot_free():
                pltpu.make_async_copy(
                    shared.at[sid, u],
                    out_hbm.at[pl.ds(off - S * CHUNK, CHUNK)],
                    osem.at[u]).wait()

            pltpu.async_copy(rows_v.at[u], shared.at[sid, u], psem.at[u])

            u2 = (u - D) % M
            c2 = c - D

            @pl.when(c2 >= 0)
            def _drain():
                off2 = base + c2 * CHUNK
                pltpu.make_async_copy(
                    rows_v.at[u2], shared.at[sid, u2], psem.at[u2]).wait()
                pltpu.async_copy(
                    shared.at[sid, u2],
                    out_hbm.at[pl.ds(off2, CHUNK)], osem.at[u2])

                @pl.when(c2 + M < NCHUNK)
                def _refill():
                    pltpu.async_copy(
                        table_hbm.at[idx_v.at[pl.ds((c2 + M) * CHUNK, CHUNK)]],
                        rows_v.at[u2], gsem.at[u2])
        return carry

    lax.fori_loop(0, NCHUNK // M, outer, 0)

    for c2 in range(NCHUNK - D, NCHUNK):
        u2 = c2 % M
        off2 = base + c2 * CHUNK
        pltpu.make_async_copy(
            rows_v.at[u2], shared.at[sid, u2], psem.at[u2]).wait()
        pltpu.async_copy(
            shared.at[sid, u2], out_hbm.at[pl.ds(off2, CHUNK)], osem.at[u2])

    for c2 in range(NCHUNK - S, NCHUNK):
        u2 = c2 % S
        off2 = base + c2 * CHUNK
        pltpu.make_async_copy(
            shared.at[sid, u2], out_hbm.at[pl.ds(off2, CHUNK)],
            osem.at[u2]).wait()


def kernel(inputs, embedding):
    idx = inputs.reshape(-1).astype(jnp.int32)
    out = _gather(idx, embedding)
    return out.reshape(BATCH, HIST, EMB)
